# Initial kernel scaffold; baseline (speedup 1.0000x reference)
import jax
import jax.numpy as jnp
from jax.experimental import pallas as pl


def kernel(inputs):
    raise NotImplementedError("write your pallas kernel here")


# speedup vs baseline: 1.7720x; 1.7720x over previous
import functools

import numpy as np
import jax
import jax.numpy as jnp
from jax import lax
from jax.experimental import pallas as pl
from jax.experimental.pallas import tpu as pltpu
from jax.experimental.pallas import tpu_sc as plsc

_KH, _KW, _SH, _SW = 3, 3, 2, 2


@functools.cache
def _build_gather(B, H, W, C, D):
    Hout = (H - _KH) // _SH + 1
    Wout = (W - _KW) // _SW + 1
    CD = C * D
    R = B * Hout * Wout * _KH * _KW

    bb = np.arange(B).reshape(B, 1, 1, 1, 1)
    hh = np.arange(Hout).reshape(1, Hout, 1, 1, 1)
    ww = np.arange(Wout).reshape(1, 1, Wout, 1, 1)
    kh = np.arange(_KH).reshape(1, 1, 1, _KH, 1)
    kw = np.arange(_KW).reshape(1, 1, 1, 1, _KW)
    idx = (bb * (H * W) + (_SH * hh + kh) * W + (_SW * ww + kw)).astype(np.int32)
    idx = idx.reshape(R)

    urows = 8
    for cand in range(8, 129, 8):
        if R % cand == 0 and cand * CD * 4 <= 200_000:
            urows = cand
    nunits = R // urows
    idx1 = jnp.asarray(idx)

    info = plsc.get_sparse_core_info()
    nc, ns = info.num_cores, info.num_subcores
    nw = nc * ns
    iters = -(-nunits // nw)

    mesh = plsc.VectorSubcoreMesh(core_axis_name="c", subcore_axis_name="s")

    @functools.partial(
        pl.kernel,
        mesh=mesh,
        out_type=jax.ShapeDtypeStruct((R, CD), jnp.float32),
        scratch_types=[
            pltpu.VMEM((urows,), jnp.int32),
            pltpu.VMEM((urows, CD), jnp.float32),
            pltpu.SemaphoreType.DMA,
        ],
    )
    def gather_kernel(x_hbm, idx_hbm, out_hbm, idx_v, rows_v, sem):
        wid = lax.axis_index("s") * nc + lax.axis_index("c")

        def body(i, carry):
            u = i * nw + wid

            @pl.when(u < nunits)
            def _():
                pltpu.sync_copy(idx_hbm.at[pl.ds(u * urows, urows)], idx_v)
                pltpu.async_copy(x_hbm.at[idx_v], rows_v, sem).wait()
                pltpu.sync_copy(rows_v, out_hbm.at[pl.ds(u * urows, urows)])

            return carry

        lax.fori_loop(0, iters, body, 0)

    def run(inputs):
        x2 = inputs.reshape(B * H * W, CD)
        out = gather_kernel(x2, idx1)
        return out.reshape(-1, Hout * _KH * _KW * C, D), Hout

    return run


def kernel(inputs):
    B, H, W, C, D = inputs.shape
    return _build_gather(B, H, W, C, D)(inputs)
